# baseline (speedup 1.0000x reference)
import jax
import jax.numpy as jnp
from jax import lax
from jax.experimental import pallas as pl
from jax.experimental.pallas import tpu as pltpu
from jax.experimental.pallas import tpu_sc as plsc

N = 10000
NN = 10240
E = 320000
EP = 327680
D = 128
G = 64
NC, NS = 2, 16
NW = NC * NS
EPW = EP // NW
CH = 128
NCH = EP // CH
CT = NCH // NS
QC = CT // 4
EPAD = NCH * CH
RB = 1024
NRB = NN // RB
PAD_DST = 10008

_mesh = plsc.VectorSubcoreMesh(
    core_axis_name="c", subcore_axis_name="s", num_cores=NC, num_subcores=NS
)


def _deg_body(dst_hbm, out_hbm, degl, dbuf, sem):
    wid = lax.axis_index("s") * NC + lax.axis_index("c")
    zero16 = jnp.zeros((16,), jnp.float32)

    def zero(i, _):
        degl[pl.ds(i * 16, 16)] = zero16
        return 0

    lax.fori_loop(0, NN // 16, zero, 0)
    pltpu.async_copy(dst_hbm.at[pl.ds(wid * EPW, EPW)], dbuf, sem).wait()
    ones16 = jnp.ones((16,), jnp.float32)

    def body(j, _):
        idx = dbuf[pl.ds(j * 16, 16)]
        plsc.addupdate_scatter(degl, [idx], ones16)
        return 0

    lax.fori_loop(0, EPW // 16, body, 0)
    pltpu.sync_copy(degl, out_hbm.at[wid])


_deg = pl.kernel(
    _deg_body,
    out_type=jax.ShapeDtypeStruct((NW, NN), jnp.float32),
    mesh=_mesh,
    scratch_types=[
        pltpu.VMEM((NN,), jnp.float32),
        pltpu.VMEM((EPW,), jnp.int32),
        pltpu.SemaphoreType.DMA,
    ],
    compiler_params=pltpu.CompilerParams(needs_layout_passes=False),
)


def _scat_body(ts_hbm, src_hbm, dst_hbm, zeros_hbm, out_hbm, acc, rows, sbuf, dbuf, sem):
    cid = lax.axis_index("c")
    sid = lax.axis_index("s")
    on0 = cid == 0

    @pl.when(on0)
    def _():
        zbase = sid * (NN // NS)
        pltpu.sync_copy(zeros_hbm.at[pl.ds(zbase, NN // NS)], acc.at[pl.ds(zbase, NN // NS)])

    plsc.subcore_barrier()

    cbase = jnp.where(on0, sid * CT, 0)
    qc = jnp.where(on0, QC, 0)
    for h in range(4):
        hbase = pl.multiple_of(cbase + h * qc, 8)
        pltpu.sync_copy(src_hbm.at[pl.ds(hbase, QC)], sbuf)
        pltpu.sync_copy(dst_hbm.at[pl.ds(hbase, QC)], dbuf)

        @pl.when(on0)
        def _():
            pltpu.async_copy(ts_hbm.at[sbuf.at[0]], rows.at[0], sem)

        def body(j, _):
            par = lax.rem(j, 2)
            pltpu.make_async_copy(ts_hbm.at[sbuf.at[j]], rows.at[par], sem).wait()

            @pl.when(j + 1 < qc)
            def _():
                pltpu.async_copy(ts_hbm.at[sbuf.at[j + 1]], rows.at[1 - par], sem)

            pltpu.sync_copy(rows.at[par], acc.at[dbuf.at[j]], add=True)
            return 0

        lax.fori_loop(0, qc, body, 0)
    plsc.subcore_barrier()
    rpt = NN // NS

    @pl.when(on0)
    def _():
        pltpu.sync_copy(acc.at[pl.ds(sid * rpt, rpt)], out_hbm.at[pl.ds(sid * rpt, rpt)])


_scat = pl.kernel(
    _scat_body,
    out_type=jax.ShapeDtypeStruct((NN, D), jnp.float32),
    mesh=_mesh,
    scratch_types=[
        pltpu.VMEM_SHARED((NN, D), jnp.float32),
        pltpu.VMEM((2, CH, D), jnp.float32),
        pltpu.VMEM((QC, CH), jnp.int32),
        pltpu.VMEM((QC, CH), jnp.int32),
        pltpu.SemaphoreType.DMA,
    ],
)


def _l1_body(degp, x, w, dinv_o, ts_o):
    deg = jnp.sum(degp[...], axis=0) + 1.0
    dinv = lax.rsqrt(deg)
    dinv_o[...] = dinv[:, None]
    ts_o[...] = jnp.dot(x[...], w[...], preferred_element_type=jnp.float32) * dinv[:, None]


_l1 = pl.pallas_call(
    _l1_body,
    grid=(NRB,),
    in_specs=[
        pl.BlockSpec((NW, RB), lambda i: (0, i)),
        pl.BlockSpec((RB, D), lambda i: (i, 0)),
        pl.BlockSpec((D, D), lambda i: (0, 0)),
    ],
    out_specs=[
        pl.BlockSpec((RB, 1), lambda i: (i, 0)),
        pl.BlockSpec((RB, D), lambda i: (i, 0)),
    ],
    out_shape=[
        jax.ShapeDtypeStruct((NN, 1), jnp.float32),
        jax.ShapeDtypeStruct((NN, D), jnp.float32),
    ],
)


def _mid_body(S, ts, dinv, b, w, out):
    h = jnp.maximum(dinv[...] * (S[...] + ts[...]) + b[...], 0.0)
    out[...] = jnp.dot(h, w[...], preferred_element_type=jnp.float32) * dinv[...]


_mid = pl.pallas_call(
    _mid_body,
    grid=(NRB,),
    in_specs=[
        pl.BlockSpec((RB, D), lambda i: (i, 0)),
        pl.BlockSpec((RB, D), lambda i: (i, 0)),
        pl.BlockSpec((RB, 1), lambda i: (i, 0)),
        pl.BlockSpec((1, D), lambda i: (0, 0)),
        pl.BlockSpec((D, D), lambda i: (0, 0)),
    ],
    out_specs=pl.BlockSpec((RB, D), lambda i: (i, 0)),
    out_shape=jax.ShapeDtypeStruct((NN, D), jnp.float32),
)


def _fin_body(S, ts, dinv, b, bidx, w1, b1, w2, b2, out, pooled, cnt):
    k = pl.program_id(0)
    h = jnp.maximum(dinv[...] * (S[...] + ts[...]) + b[...], 0.0)
    gid = lax.broadcasted_iota(jnp.int32, (G, 1), 0)
    mask = (bidx[...] == gid).astype(jnp.float32)
    pm = jnp.dot(mask, h, preferred_element_type=jnp.float32)
    cm = jnp.sum(mask, axis=1, keepdims=True)

    @pl.when(k == 0)
    def _():
        pooled[...] = pm
        cnt[...] = cm

    @pl.when(k > 0)
    def _():
        pooled[...] += pm
        cnt[...] += cm

    @pl.when(k == NRB - 1)
    def _():
        pool = pooled[...] / jnp.maximum(cnt[...], 1.0)
        g = jnp.maximum(
            jnp.dot(pool, w1[...], preferred_element_type=jnp.float32) + b1[...], 0.0
        )
        out[...] = jnp.dot(g, w2[...], preferred_element_type=jnp.float32) + b2[...]


_fin = pl.pallas_call(
    _fin_body,
    grid=(NRB,),
    in_specs=[
        pl.BlockSpec((RB, D), lambda i: (i, 0)),
        pl.BlockSpec((RB, D), lambda i: (i, 0)),
        pl.BlockSpec((RB, 1), lambda i: (i, 0)),
        pl.BlockSpec((1, D), lambda i: (0, 0)),
        pl.BlockSpec((1, RB), lambda i: (0, i)),
        pl.BlockSpec((D, D), lambda i: (0, 0)),
        pl.BlockSpec((1, D), lambda i: (0, 0)),
        pl.BlockSpec((D, 1), lambda i: (0, 0)),
        pl.BlockSpec((1, 1), lambda i: (0, 0)),
    ],
    out_specs=pl.BlockSpec((G, 1), lambda i: (0, 0)),
    out_shape=jax.ShapeDtypeStruct((G, 1), jnp.float32),
    scratch_shapes=[
        pltpu.VMEM((G, D), jnp.float32),
        pltpu.VMEM((G, 1), jnp.float32),
    ],
)


def kernel(x, edge_index, batch_idx, W1, b1, Ws, bs, lin1_W, lin1_b, lin2_W, lin2_b):
    src = edge_index[0].astype(jnp.int32)
    dst = edge_index[1].astype(jnp.int32)
    pad = EPAD - E
    srcp = jnp.concatenate([src, jnp.zeros((pad,), jnp.int32)])
    dstp = jnp.concatenate([dst, jnp.full((pad,), PAD_DST, jnp.int32)])
    xp = jnp.pad(x, ((0, NN - N), (0, 0)))
    bidxp = jnp.pad(batch_idx.astype(jnp.int32), (0, NN - N), constant_values=G)

    src2 = srcp.reshape(EPAD // CH, CH)
    dst2 = dstp.reshape(EPAD // CH, CH)

    degp = _deg(dstp[:EP])
    dinv, ts = _l1(degp, xp, W1)
    zrows = jnp.zeros((NN, D), jnp.float32)
    S = _scat(ts, src2, dst2, zrows)
    ts = _mid(S, ts, dinv, b1.reshape(1, D), Ws[0])
    S = _scat(ts, src2, dst2, zrows)
    ts = _mid(S, ts, dinv, bs[0].reshape(1, D), Ws[1])
    S = _scat(ts, src2, dst2, zrows)
    out = _fin(
        S, ts, dinv, bs[1].reshape(1, D),
        bidxp.reshape(1, NN),
        lin1_W, lin1_b.reshape(1, D), lin2_W, lin2_b.reshape(1, 1),
    )
    return out


# speedup vs baseline: 1.4477x; 1.4477x over previous
import jax
import jax.numpy as jnp
from jax import lax
from jax.experimental import pallas as pl
from jax.experimental.pallas import tpu as pltpu
from jax.experimental.pallas import tpu_sc as plsc

N = 10000
NN = 10240
E = 320000
EP = 327680
D = 128
G = 64
NC, NS = 2, 16
NW = NC * NS
EPW = EP // NW
CH = 128
NCH = EP // CH
CA = 128
CB = 32
QMAX = CA // 4
EPAD = (NCH + 64) * CH
ZR = 32
RB = 1024
NRB = NN // RB
PAD_DST = 10008

_mesh = plsc.VectorSubcoreMesh(
    core_axis_name="c", subcore_axis_name="s", num_cores=NC, num_subcores=NS
)


def _deg_body(dst_hbm, out_hbm, degl, dbuf, sem):
    wid = lax.axis_index("s") * NC + lax.axis_index("c")
    zero16 = jnp.zeros((16,), jnp.float32)

    def zero(i, _):
        degl[pl.ds(i * 16, 16)] = zero16
        return 0

    lax.fori_loop(0, NN // 16, zero, 0)
    pltpu.async_copy(dst_hbm.at[pl.ds(wid * EPW, EPW)], dbuf, sem).wait()
    ones16 = jnp.ones((16,), jnp.float32)

    def body(j, _):
        idx = dbuf[pl.ds(j * 16, 16)]
        plsc.addupdate_scatter(degl, [idx], ones16)
        return 0

    lax.fori_loop(0, EPW // 16, body, 0)
    pltpu.sync_copy(degl, out_hbm.at[wid])


_deg = pl.kernel(
    _deg_body,
    out_type=jax.ShapeDtypeStruct((NW, NN), jnp.float32),
    mesh=_mesh,
    scratch_types=[
        pltpu.VMEM((NN,), jnp.float32),
        pltpu.VMEM((EPW,), jnp.int32),
        pltpu.SemaphoreType.DMA,
    ],
    compiler_params=pltpu.CompilerParams(needs_layout_passes=False),
)


def _scat_body(ts_hbm, src_hbm, dst_hbm, out_hbm, acc, rows, sbuf, dbuf, zbuf, sem, zsem):
    cid = lax.axis_index("c")
    sid = lax.axis_index("s")
    zero16 = jnp.zeros((16,), jnp.float32)

    def zfill(i, _):
        for j in range(D // 16):
            zbuf[i, pl.ds(j * 16, 16)] = zero16
        return 0

    lax.fori_loop(0, ZR, zfill, 0)
    zbase = sid * (NN // NS)
    nz = (NN // NS) // ZR

    def zstart(i, _):
        pltpu.async_copy(zbuf, acc.at[pl.ds(zbase + i * ZR, ZR)], zsem)
        return 0

    lax.fori_loop(0, nz, zstart, 0)

    def zdrain(i, _):
        pltpu.make_async_copy(zbuf, acc.at[pl.ds(zbase + i * ZR, ZR)], zsem).wait()
        return 0

    lax.fori_loop(0, nz, zdrain, 0)
    plsc.subcore_barrier()

    count = jnp.where(cid == 0, CA, CB)
    hc = count // 4
    cbase = jnp.where(cid == 0, sid * CA, NS * CA + sid * CB)
    for h in range(4):
        hbase = pl.multiple_of(cbase + h * hc, 8)
        pltpu.sync_copy(src_hbm.at[pl.ds(hbase, QMAX)], sbuf)
        pltpu.sync_copy(dst_hbm.at[pl.ds(hbase, QMAX)], dbuf)
        pltpu.async_copy(ts_hbm.at[sbuf.at[0]], rows.at[0], sem)

        def body(j, _):
            par = lax.rem(j, 2)
            pltpu.make_async_copy(ts_hbm.at[sbuf.at[j]], rows.at[par], sem).wait()

            @pl.when(j + 1 < hc)
            def _():
                pltpu.async_copy(ts_hbm.at[sbuf.at[j + 1]], rows.at[1 - par], sem)

            pltpu.sync_copy(rows.at[par], acc.at[dbuf.at[j]], add=True)
            return 0

        lax.fori_loop(0, hc, body, 0)
    plsc.subcore_barrier()
    rpt = NN // NS
    pltpu.sync_copy(acc.at[pl.ds(sid * rpt, rpt)], out_hbm.at[cid, pl.ds(sid * rpt, rpt)])


_scat = pl.kernel(
    _scat_body,
    out_type=jax.ShapeDtypeStruct((NC, NN, D), jnp.float32),
    mesh=_mesh,
    scratch_types=[
        pltpu.VMEM_SHARED((NN, D), jnp.float32),
        pltpu.VMEM((2, CH, D), jnp.float32),
        pltpu.VMEM((QMAX, CH), jnp.int32),
        pltpu.VMEM((QMAX, CH), jnp.int32),
        pltpu.VMEM((ZR, D), jnp.float32),
        pltpu.SemaphoreType.DMA,
        pltpu.SemaphoreType.DMA,
    ],
)


def _l1_body(degp, x, w, dinv_o, ts_o):
    deg = jnp.sum(degp[...], axis=0) + 1.0
    dinv = lax.rsqrt(deg)
    dinv_o[...] = dinv[:, None]
    ts_o[...] = jnp.dot(x[...], w[...], preferred_element_type=jnp.float32) * dinv[:, None]


_l1 = pl.pallas_call(
    _l1_body,
    grid=(NRB,),
    in_specs=[
        pl.BlockSpec((NW, RB), lambda i: (0, i)),
        pl.BlockSpec((RB, D), lambda i: (i, 0)),
        pl.BlockSpec((D, D), lambda i: (0, 0)),
    ],
    out_specs=[
        pl.BlockSpec((RB, 1), lambda i: (i, 0)),
        pl.BlockSpec((RB, D), lambda i: (i, 0)),
    ],
    out_shape=[
        jax.ShapeDtypeStruct((NN, 1), jnp.float32),
        jax.ShapeDtypeStruct((NN, D), jnp.float32),
    ],
)


def _mid_body(S, ts, dinv, b, w, out):
    h = jnp.maximum(dinv[...] * (S[0] + S[1] + ts[...]) + b[...], 0.0)
    out[...] = jnp.dot(h, w[...], preferred_element_type=jnp.float32) * dinv[...]


_mid = pl.pallas_call(
    _mid_body,
    grid=(NRB,),
    in_specs=[
        pl.BlockSpec((NC, RB, D), lambda i: (0, i, 0)),
        pl.BlockSpec((RB, D), lambda i: (i, 0)),
        pl.BlockSpec((RB, 1), lambda i: (i, 0)),
        pl.BlockSpec((1, D), lambda i: (0, 0)),
        pl.BlockSpec((D, D), lambda i: (0, 0)),
    ],
    out_specs=pl.BlockSpec((RB, D), lambda i: (i, 0)),
    out_shape=jax.ShapeDtypeStruct((NN, D), jnp.float32),
)


def _fin_body(S, ts, dinv, b, bidx, w1, b1, w2, b2, out, pooled, cnt):
    k = pl.program_id(0)
    h = jnp.maximum(dinv[...] * (S[0] + S[1] + ts[...]) + b[...], 0.0)
    gid = lax.broadcasted_iota(jnp.int32, (G, 1), 0)
    mask = (bidx[...] == gid).astype(jnp.float32)
    pm = jnp.dot(mask, h, preferred_element_type=jnp.float32)
    cm = jnp.sum(mask, axis=1, keepdims=True)

    @pl.when(k == 0)
    def _():
        pooled[...] = pm
        cnt[...] = cm

    @pl.when(k > 0)
    def _():
        pooled[...] += pm
        cnt[...] += cm

    @pl.when(k == NRB - 1)
    def _():
        pool = pooled[...] / jnp.maximum(cnt[...], 1.0)
        g = jnp.maximum(
            jnp.dot(pool, w1[...], preferred_element_type=jnp.float32) + b1[...], 0.0
        )
        out[...] = jnp.dot(g, w2[...], preferred_element_type=jnp.float32) + b2[...]


_fin = pl.pallas_call(
    _fin_body,
    grid=(NRB,),
    in_specs=[
        pl.BlockSpec((NC, RB, D), lambda i: (0, i, 0)),
        pl.BlockSpec((RB, D), lambda i: (i, 0)),
        pl.BlockSpec((RB, 1), lambda i: (i, 0)),
        pl.BlockSpec((1, D), lambda i: (0, 0)),
        pl.BlockSpec((1, RB), lambda i: (0, i)),
        pl.BlockSpec((D, D), lambda i: (0, 0)),
        pl.BlockSpec((1, D), lambda i: (0, 0)),
        pl.BlockSpec((D, 1), lambda i: (0, 0)),
        pl.BlockSpec((1, 1), lambda i: (0, 0)),
    ],
    out_specs=pl.BlockSpec((G, 1), lambda i: (0, 0)),
    out_shape=jax.ShapeDtypeStruct((G, 1), jnp.float32),
    scratch_shapes=[
        pltpu.VMEM((G, D), jnp.float32),
        pltpu.VMEM((G, 1), jnp.float32),
    ],
)


def kernel(x, edge_index, batch_idx, W1, b1, Ws, bs, lin1_W, lin1_b, lin2_W, lin2_b):
    src = edge_index[0].astype(jnp.int32)
    dst = edge_index[1].astype(jnp.int32)
    pad = EPAD - E
    srcp = jnp.concatenate([src, jnp.zeros((pad,), jnp.int32)])
    dstp = jnp.concatenate([dst, jnp.full((pad,), PAD_DST, jnp.int32)])
    xp = jnp.pad(x, ((0, NN - N), (0, 0)))
    bidxp = jnp.pad(batch_idx.astype(jnp.int32), (0, NN - N), constant_values=G)

    src2 = srcp.reshape(EPAD // CH, CH)
    dst2 = dstp.reshape(EPAD // CH, CH)

    degp = _deg(dstp[:EP])
    dinv, ts = _l1(degp, xp, W1)
    S = _scat(ts, src2, dst2)
    ts = _mid(S, ts, dinv, b1.reshape(1, D), Ws[0])
    S = _scat(ts, src2, dst2)
    ts = _mid(S, ts, dinv, bs[0].reshape(1, D), Ws[1])
    S = _scat(ts, src2, dst2)
    out = _fin(
        S, ts, dinv, bs[1].reshape(1, D),
        bidxp.reshape(1, NN),
        lin1_W, lin1_b.reshape(1, D), lin2_W, lin2_b.reshape(1, 1),
    )
    return out


# speedup vs baseline: 1.5041x; 1.0390x over previous
import jax
import jax.numpy as jnp
from jax import lax
from jax.experimental import pallas as pl
from jax.experimental.pallas import tpu as pltpu
from jax.experimental.pallas import tpu_sc as plsc

N = 10000
NN = 10240
E = 320000
EP = 327680
D = 128
G = 64
NC, NS = 2, 16
NW = NC * NS
EPW = EP // NW
CH = 128
NCH = EP // CH
CA = 128
CB = 32
QMAX = CA // 4
EPAD = (NCH + 64) * CH
RB = 1024
NRB = NN // RB
PAD_DST = 10008

_mesh = plsc.VectorSubcoreMesh(
    core_axis_name="c", subcore_axis_name="s", num_cores=NC, num_subcores=NS
)


def _deg_body(dst_hbm, out_hbm, degl, dbuf, sem):
    wid = lax.axis_index("s") * NC + lax.axis_index("c")
    zero16 = jnp.zeros((16,), jnp.float32)

    def zero(i, _):
        degl[pl.ds(i * 16, 16)] = zero16
        return 0

    lax.fori_loop(0, NN // 16, zero, 0)
    pltpu.async_copy(dst_hbm.at[pl.ds(wid * EPW, EPW)], dbuf, sem).wait()
    ones16 = jnp.ones((16,), jnp.float32)

    def body(j, _):
        idx = dbuf[pl.ds(j * 16, 16)]
        plsc.addupdate_scatter(degl, [idx], ones16)
        return 0

    lax.fori_loop(0, EPW // 16, body, 0)
    pltpu.sync_copy(degl, out_hbm.at[wid])


_deg = pl.kernel(
    _deg_body,
    out_type=jax.ShapeDtypeStruct((NW, NN), jnp.float32),
    mesh=_mesh,
    scratch_types=[
        pltpu.VMEM((NN,), jnp.float32),
        pltpu.VMEM((EPW,), jnp.int32),
        pltpu.SemaphoreType.DMA,
    ],
    compiler_params=pltpu.CompilerParams(needs_layout_passes=False),
)


def _scat_body(ts_hbm, src_hbm, dst_hbm, zeros_hbm, out_hbm, acc, rows, sbuf, dbuf, sem):
    cid = lax.axis_index("c")
    sid = lax.axis_index("s")
    zbase = sid * (NN // NS)
    pltpu.sync_copy(zeros_hbm.at[pl.ds(zbase, NN // NS)], acc.at[pl.ds(zbase, NN // NS)])
    plsc.subcore_barrier()

    count = jnp.where(cid == 0, CA, CB)
    hc = count // 4
    cbase = jnp.where(cid == 0, sid * CA, NS * CA + sid * CB)
    for h in range(4):
        hbase = pl.multiple_of(cbase + h * hc, 8)
        pltpu.sync_copy(src_hbm.at[pl.ds(hbase, QMAX)], sbuf)
        pltpu.sync_copy(dst_hbm.at[pl.ds(hbase, QMAX)], dbuf)
        pltpu.async_copy(ts_hbm.at[sbuf.at[0]], rows.at[0], sem)

        def body(j, _):
            par = lax.rem(j, 2)
            pltpu.make_async_copy(ts_hbm.at[sbuf.at[j]], rows.at[par], sem).wait()

            @pl.when(j + 1 < hc)
            def _():
                pltpu.async_copy(ts_hbm.at[sbuf.at[j + 1]], rows.at[1 - par], sem)

            pltpu.sync_copy(rows.at[par], acc.at[dbuf.at[j]], add=True)
            return 0

        lax.fori_loop(0, hc, body, 0)
    plsc.subcore_barrier()
    rpt = NN // NS
    pltpu.sync_copy(acc.at[pl.ds(sid * rpt, rpt)], out_hbm.at[cid, pl.ds(sid * rpt, rpt)])


_scat = pl.kernel(
    _scat_body,
    out_type=jax.ShapeDtypeStruct((NC, NN, D), jnp.float32),
    mesh=_mesh,
    scratch_types=[
        pltpu.VMEM_SHARED((NN, D), jnp.float32),
        pltpu.VMEM((2, CH, D), jnp.float32),
        pltpu.VMEM((QMAX, CH), jnp.int32),
        pltpu.VMEM((QMAX, CH), jnp.int32),
        pltpu.SemaphoreType.DMA,
    ],
)


def _l1_body(degp, x, w, dinv_o, ts_o):
    deg = jnp.sum(degp[...], axis=0) + 1.0
    dinv = lax.rsqrt(deg)
    dinv_o[...] = dinv[:, None]
    ts_o[...] = jnp.dot(x[...], w[...], preferred_element_type=jnp.float32) * dinv[:, None]


_l1 = pl.pallas_call(
    _l1_body,
    grid=(NRB,),
    in_specs=[
        pl.BlockSpec((NW, RB), lambda i: (0, i)),
        pl.BlockSpec((RB, D), lambda i: (i, 0)),
        pl.BlockSpec((D, D), lambda i: (0, 0)),
    ],
    out_specs=[
        pl.BlockSpec((RB, 1), lambda i: (i, 0)),
        pl.BlockSpec((RB, D), lambda i: (i, 0)),
    ],
    out_shape=[
        jax.ShapeDtypeStruct((NN, 1), jnp.float32),
        jax.ShapeDtypeStruct((NN, D), jnp.float32),
    ],
)


def _mid_body(S, ts, dinv, b, w, out):
    h = jnp.maximum(dinv[...] * (S[0] + S[1] + ts[...]) + b[...], 0.0)
    out[...] = jnp.dot(h, w[...], preferred_element_type=jnp.float32) * dinv[...]


_mid = pl.pallas_call(
    _mid_body,
    grid=(NRB,),
    in_specs=[
        pl.BlockSpec((NC, RB, D), lambda i: (0, i, 0)),
        pl.BlockSpec((RB, D), lambda i: (i, 0)),
        pl.BlockSpec((RB, 1), lambda i: (i, 0)),
        pl.BlockSpec((1, D), lambda i: (0, 0)),
        pl.BlockSpec((D, D), lambda i: (0, 0)),
    ],
    out_specs=pl.BlockSpec((RB, D), lambda i: (i, 0)),
    out_shape=jax.ShapeDtypeStruct((NN, D), jnp.float32),
)


def _fin_body(S, ts, dinv, b, bidx, w1, b1, w2, b2, out, pooled, cnt):
    k = pl.program_id(0)
    h = jnp.maximum(dinv[...] * (S[0] + S[1] + ts[...]) + b[...], 0.0)
    gid = lax.broadcasted_iota(jnp.int32, (G, 1), 0)
    mask = (bidx[...] == gid).astype(jnp.float32)
    pm = jnp.dot(mask, h, preferred_element_type=jnp.float32)
    cm = jnp.sum(mask, axis=1, keepdims=True)

    @pl.when(k == 0)
    def _():
        pooled[...] = pm
        cnt[...] = cm

    @pl.when(k > 0)
    def _():
        pooled[...] += pm
        cnt[...] += cm

    @pl.when(k == NRB - 1)
    def _():
        pool = pooled[...] / jnp.maximum(cnt[...], 1.0)
        g = jnp.maximum(
            jnp.dot(pool, w1[...], preferred_element_type=jnp.float32) + b1[...], 0.0
        )
        out[...] = jnp.dot(g, w2[...], preferred_element_type=jnp.float32) + b2[...]


_fin = pl.pallas_call(
    _fin_body,
    grid=(NRB,),
    in_specs=[
        pl.BlockSpec((NC, RB, D), lambda i: (0, i, 0)),
        pl.BlockSpec((RB, D), lambda i: (i, 0)),
        pl.BlockSpec((RB, 1), lambda i: (i, 0)),
        pl.BlockSpec((1, D), lambda i: (0, 0)),
        pl.BlockSpec((1, RB), lambda i: (0, i)),
        pl.BlockSpec((D, D), lambda i: (0, 0)),
        pl.BlockSpec((1, D), lambda i: (0, 0)),
        pl.BlockSpec((D, 1), lambda i: (0, 0)),
        pl.BlockSpec((1, 1), lambda i: (0, 0)),
    ],
    out_specs=pl.BlockSpec((G, 1), lambda i: (0, 0)),
    out_shape=jax.ShapeDtypeStruct((G, 1), jnp.float32),
    scratch_shapes=[
        pltpu.VMEM((G, D), jnp.float32),
        pltpu.VMEM((G, 1), jnp.float32),
    ],
)


def kernel(x, edge_index, batch_idx, W1, b1, Ws, bs, lin1_W, lin1_b, lin2_W, lin2_b):
    src = edge_index[0].astype(jnp.int32)
    dst = edge_index[1].astype(jnp.int32)
    pad = EPAD - E
    srcp = jnp.concatenate([src, jnp.zeros((pad,), jnp.int32)])
    dstp = jnp.concatenate([dst, jnp.full((pad,), PAD_DST, jnp.int32)])
    xp = jnp.pad(x, ((0, NN - N), (0, 0)))
    bidxp = jnp.pad(batch_idx.astype(jnp.int32), (0, NN - N), constant_values=G)

    src2 = srcp.reshape(EPAD // CH, CH)
    dst2 = dstp.reshape(EPAD // CH, CH)

    degp = _deg(dstp[:EP])
    dinv, ts = _l1(degp, xp, W1)
    zrows = jnp.zeros((NN, D), jnp.float32)
    S = _scat(ts, src2, dst2, zrows)
    ts = _mid(S, ts, dinv, b1.reshape(1, D), Ws[0])
    S = _scat(ts, src2, dst2, zrows)
    ts = _mid(S, ts, dinv, bs[0].reshape(1, D), Ws[1])
    S = _scat(ts, src2, dst2, zrows)
    out = _fin(
        S, ts, dinv, bs[1].reshape(1, D),
        bidxp.reshape(1, NN),
        lin1_W, lin1_b.reshape(1, D), lin2_W, lin2_b.reshape(1, 1),
    )
    return out
